# baseline (speedup 1.0000x reference)
import jax
import jax.numpy as jnp
from jax.experimental import pallas as pl

_MAX_PATH_DISTANCE = 5.0
_ROW_BLOCK = 256


def _edge_encoding_block(ev_ref, w_ref, o_ref):
    s = jnp.sum(ev_ref[...]) / ev_ref.size
    o_ref[...] = jnp.nan_to_num(
        jnp.minimum(w_ref[...], jnp.float32(_MAX_PATH_DISTANCE)) * s
    )


def kernel(x, edge_attr, weights, edge_vector):
    n_rows, n_cols = weights.shape
    blk = _ROW_BLOCK
    grid = (pl.cdiv(n_rows, blk),)
    return pl.pallas_call(
        _edge_encoding_block,
        grid=grid,
        in_specs=[
            pl.BlockSpec(edge_vector.shape, lambda i: (0, 0)),
            pl.BlockSpec((blk, n_cols), lambda i: (i, 0)),
        ],
        out_specs=pl.BlockSpec((blk, n_cols), lambda i: (i, 0)),
        out_shape=jax.ShapeDtypeStruct((n_rows, n_cols), jnp.float32),
    )(edge_vector, weights)
